# baseline (speedup 1.0000x reference)
import jax
import jax.numpy as jnp
from jax import lax
from jax.experimental import pallas as pl
from jax.experimental.pallas import tpu as pltpu

B, C, H, W = 64, 384, 32, 32
NROW = B * H * W
RB = 2048
CM = 128
CG = C - CM


def _body(x_ref, idx_ref, o_ref):
    idx = idx_ref[...]
    x = x_ref[...]

    rows_k = lax.broadcasted_iota(jnp.int32, (C, CM), 0)
    P = (rows_k == idx[None, :CM]).astype(jnp.float32)
    o_ref[:, :CM] = jnp.dot(
        x, P,
        preferred_element_type=jnp.float32,
        precision=lax.Precision.HIGHEST,
    )

    idx_g = idx[CM:]
    off2d = jnp.broadcast_to((idx_g % 128)[None, :], (RB, CG))
    sel = (idx_g // 128)[None, :]
    g0 = jnp.take_along_axis(x[:, 0:128], off2d, axis=1)
    g1 = jnp.take_along_axis(x[:, 128:256], off2d, axis=1)
    g2 = jnp.take_along_axis(x[:, 256:384], off2d, axis=1)
    m1 = (sel == 1).astype(jnp.float32)
    m2 = (sel == 2).astype(jnp.float32)
    o_ref[:, CM:] = g0 + m1 * (g1 - g0) + m2 * (g2 - g0)


@jax.jit
def _tc_gather(xT, idx):
    return pl.pallas_call(
        _body,
        grid=(NROW // RB,),
        in_specs=[
            pl.BlockSpec((RB, C), lambda i: (i, 0)),
            pl.BlockSpec((C,), lambda i: (0,)),
        ],
        out_specs=pl.BlockSpec((RB, C), lambda i: (i, 0)),
        out_shape=jax.ShapeDtypeStruct((NROW, C), jnp.float32),
    )(xT, idx)


def kernel(x, indices):
    idx = indices.astype(jnp.int32)
    xT = x.transpose(0, 2, 3, 1).reshape(NROW, C)
    out2 = _tc_gather(xT, idx)
    return out2.reshape(B, H, W, C).transpose(0, 3, 1, 2)


# speedup vs baseline: 1.0599x; 1.0599x over previous
import jax
import jax.numpy as jnp
from jax import lax
from jax.experimental import pallas as pl

B, C, H, W = 64, 384, 32, 32
NROW = B * H * W
RB = 4096
CM = 128
CG = C - CM


def _body(x_ref, idx_ref, o_ref):
    idx = idx_ref[...]
    x = x_ref[...]

    rows_k = lax.broadcasted_iota(jnp.int32, (C, CM), 0)
    P = (rows_k == idx[None, :CM]).astype(jnp.float32)
    o_ref[:, :CM] = jnp.dot(
        x, P,
        preferred_element_type=jnp.float32,
        precision=lax.Precision.HIGHEST,
    )

    idx_g = idx[CM:]
    off2d = jnp.broadcast_to((idx_g % 128)[None, :], (RB, CG))
    sel = (idx_g // 128)[None, :]
    g0 = jnp.take_along_axis(x[:, 0:128], off2d, axis=1)
    g1 = jnp.take_along_axis(x[:, 128:256], off2d, axis=1)
    g2 = jnp.take_along_axis(x[:, 256:384], off2d, axis=1)
    m1 = (sel == 1).astype(jnp.float32)
    m2 = (sel == 2).astype(jnp.float32)
    o_ref[:, CM:] = g0 + m1 * (g1 - g0) + m2 * (g2 - g0)


@jax.jit
def _tc_gather(xT, idx):
    return pl.pallas_call(
        _body,
        grid=(NROW // RB,),
        in_specs=[
            pl.BlockSpec((RB, C), lambda i: (i, 0)),
            pl.BlockSpec((C,), lambda i: (0,)),
        ],
        out_specs=pl.BlockSpec((RB, C), lambda i: (i, 0)),
        out_shape=jax.ShapeDtypeStruct((NROW, C), jnp.float32),
    )(xT, idx)


def kernel(x, indices):
    idx = indices.astype(jnp.int32)
    xT = x.transpose(0, 2, 3, 1).reshape(NROW, C)
    out2 = _tc_gather(xT, idx)
    return out2.reshape(B, H, W, C).transpose(0, 3, 1, 2)


# speedup vs baseline: 1.0956x; 1.0336x over previous
import jax
import jax.numpy as jnp
from jax import lax
from jax.experimental import pallas as pl

B, C, H, W = 64, 384, 32, 32
NROW = B * H * W
RB = 4096
CM = 128
CG = C - CM


def _body(x_ref, idx_ref, o_ref):
    idx = idx_ref[...]
    x = x_ref[...]

    rows_k = lax.broadcasted_iota(jnp.int32, (C, CM), 0)
    P = (rows_k == idx[None, :CM]).astype(jnp.float32)
    o_ref[:, :CM] = jnp.dot(
        x, P,
        preferred_element_type=jnp.float32,
        precision=lax.Precision.HIGHEST,
    )

    idx_g = idx[CM:]
    off2d = jnp.broadcast_to((idx_g % 128)[None, :], (RB, CG))
    sel = (idx_g // 128)[None, :]
    g0 = jnp.take_along_axis(x[:, 0:128], off2d, axis=1)
    g1 = jnp.take_along_axis(x[:, 128:256], off2d, axis=1)
    g2 = jnp.take_along_axis(x[:, 256:384], off2d, axis=1)
    sel2d = jnp.broadcast_to(sel, (RB, CG))
    o_ref[:, CM:] = jnp.where(sel2d == 0, g0, jnp.where(sel2d == 1, g1, g2))


@jax.jit
def _tc_gather(xT, idx):
    return pl.pallas_call(
        _body,
        grid=(NROW // RB,),
        in_specs=[
            pl.BlockSpec((RB, C), lambda i: (i, 0)),
            pl.BlockSpec((C,), lambda i: (0,)),
        ],
        out_specs=pl.BlockSpec((RB, C), lambda i: (i, 0)),
        out_shape=jax.ShapeDtypeStruct((NROW, C), jnp.float32),
    )(xT, idx)


def kernel(x, indices):
    idx = indices.astype(jnp.int32)
    xT = x.transpose(0, 2, 3, 1).reshape(NROW, C)
    out2 = _tc_gather(xT, idx)
    return out2.reshape(B, H, W, C).transpose(0, 3, 1, 2)
